# baseline (speedup 1.0000x reference)
import functools

import jax
import jax.numpy as jnp
from jax import lax
from jax.experimental import pallas as pl
from jax.experimental.pallas import tpu as pltpu
from jax.experimental.pallas import tpu_sc as plsc

N_NODES_C = 10000
N_EDGES_C = 320000
D_C = 128

_NC = 2
_NS = 16
_NW = _NC * _NS
_EPW = N_EDGES_C // _NW
_CH = 48
_EPW_PAD = 10080
_NCHUNK = _EPW_PAD // _CH
_NROWS_AGG = N_NODES_C + 8
_NRING = 4
_ROWS_PT = 624
_ROWS_TAIL = N_NODES_C - _NS * _ROWS_PT


def _sc_agg_body(src_hbm, dst_hbm, h_hbm, zeros_hbm, out_hbm,
                 sidx_v, dbuf0, dbuf1, dbuf2, dbuf3,
                 rows0, rows1, rows2, rows3, agg_sh,
                 gsem0, gsem1, gsem2, gsem3, dsem0, dsem1, dsem2, dsem3,
                 ssem0, ssem1, ssem2, ssem3):
    c = lax.axis_index("c")
    s = lax.axis_index("s")
    w = s * _NC + c

    rows = (rows0, rows1, rows2, rows3)
    dbuf = (dbuf0, dbuf1, dbuf2, dbuf3)
    gsem = (gsem0, gsem1, gsem2, gsem3)
    dsem = (dsem0, dsem1, dsem2, dsem3)
    ssem = (ssem0, ssem1, ssem2, ssem3)

    r0 = s * _ROWS_PT
    pltpu.sync_copy(zeros_hbm.at[pl.ds(r0, _ROWS_PT)],
                    agg_sh.at[pl.ds(r0, _ROWS_PT)])

    @pl.when(s == _NS - 1)
    def _():
        t0 = _NS * _ROWS_PT
        pltpu.sync_copy(zeros_hbm.at[pl.ds(t0, _ROWS_TAIL)],
                        agg_sh.at[pl.ds(t0, _ROWS_TAIL)])

    pltpu.sync_copy(src_hbm.at[pl.ds(w * _EPW_PAD, _EPW_PAD)], sidx_v)
    plsc.subcore_barrier()

    def gat(i, r):
        return pltpu.make_async_copy(
            h_hbm.at[sidx_v.at[pl.ds(i * _CH, _CH)]], rows[r], gsem[r])

    def didx(i, r):
        return pltpu.make_async_copy(
            dst_hbm.at[pl.ds(w * _EPW_PAD + i * _CH, _CH)], dbuf[r], dsem[r])

    def gstart(i, r):
        gat(i, r).start()
        didx(i, r).start()

    def sstart(r):
        pltpu.async_copy(rows[r], agg_sh.at[dbuf[r]], ssem[r], add=True)

    def swait(r):
        pltpu.make_async_copy(rows[r], agg_sh.at[dbuf[r]], ssem[r]).wait()

    def step(j, rj, start_next=True, wait_prev=True):
        rp = (rj + 2) % _NRING
        if wait_prev:
            swait(rp)
        if start_next:
            gstart(j + 2, rp)
        gat(j, rj).wait()
        didx(j, rj).wait()
        sstart(rj)

    gstart(0, 0)
    gstart(1, 1)
    step(0, 0, wait_prev=False)
    step(1, 1, wait_prev=False)

    def outer(g, carry):
        j = 4 * g + 2
        for u in range(_NRING):
            step(j + u, (2 + u) % _NRING)
        return carry

    lax.fori_loop(0, (_NCHUNK - 6) // 4, outer, 0)
    step(_NCHUNK - 4, 2)
    step(_NCHUNK - 3, 3)
    step(_NCHUNK - 2, 0, start_next=False)
    step(_NCHUNK - 1, 1, start_next=False)
    swait(0)
    swait(1)

    plsc.subcore_barrier()
    pltpu.sync_copy(agg_sh.at[pl.ds(r0, _ROWS_PT)],
                    out_hbm.at[c, pl.ds(r0, _ROWS_PT)])

    @pl.when(s == _NS - 1)
    def _():
        t0 = _NS * _ROWS_PT
        pltpu.sync_copy(agg_sh.at[pl.ds(t0, _ROWS_TAIL)],
                        out_hbm.at[c, pl.ds(t0, _ROWS_TAIL)])


@jax.jit
def _sc_agg(src, dst, h, zeros):
    mesh = plsc.VectorSubcoreMesh(core_axis_name="c", subcore_axis_name="s")
    k = pl.kernel(
        _sc_agg_body,
        out_type=jax.ShapeDtypeStruct((_NC, N_NODES_C, D_C), jnp.float32),
        mesh=mesh,
        scratch_types=(
            [pltpu.VMEM((_EPW_PAD,), jnp.int32)]
            + [pltpu.VMEM((_CH,), jnp.int32) for _ in range(_NRING)]
            + [pltpu.VMEM((_CH, D_C), jnp.float32) for _ in range(_NRING)]
            + [pltpu.VMEM_SHARED((_NROWS_AGG, D_C), jnp.float32)]
            + [pltpu.SemaphoreType.DMA for _ in range(3 * _NRING)]
        ),
    )
    return k(src, dst, h, zeros)


def _mlp_body(h_ref, a0_ref, a1_ref, w_ref, b_ref, eps_ref, out_ref):
    pre = (h_ref[...] * (1.0 + eps_ref[0, 0])
           + a0_ref[...] + a1_ref[...])
    y = jnp.dot(pre, w_ref[...], preferred_element_type=jnp.float32)
    out_ref[...] = jnp.maximum(y + b_ref[...], 0.0)


def _mlp_final_body(h_ref, a0_ref, a1_ref, w_ref, b_ref, eps_ref,
                    logits_ref, probs_ref):
    pre = (h_ref[...] * (1.0 + eps_ref[0, 0])
           + a0_ref[...] + a1_ref[...])
    y = jnp.dot(pre, w_ref[...], preferred_element_type=jnp.float32)
    logits = jnp.maximum(y + b_ref[...], 0.0)
    logits_ref[...] = logits
    m = jnp.max(logits, axis=-1, keepdims=True)
    e = jnp.exp(logits - m)
    probs_ref[...] = e / jnp.sum(e, axis=-1, keepdims=True)


_BN = 1000


def _row_spec():
    return pl.BlockSpec((_BN, D_C), lambda i: (i, 0))


def _full_spec(shape):
    return pl.BlockSpec(shape, lambda i: tuple(0 for _ in shape))


@jax.jit
def _tc_mlp(h, a0, a1, W, b, eps):
    return pl.pallas_call(
        _mlp_body,
        grid=(N_NODES_C // _BN,),
        in_specs=[_row_spec(), _row_spec(), _row_spec(),
                  _full_spec((D_C, D_C)), _full_spec((1, D_C)),
                  _full_spec((1, 1))],
        out_specs=_row_spec(),
        out_shape=jax.ShapeDtypeStruct((N_NODES_C, D_C), jnp.float32),
    )(h, a0, a1, W, b.reshape(1, D_C), eps.reshape(1, 1))


@jax.jit
def _tc_mlp_final(h, a0, a1, W, b, eps):
    return pl.pallas_call(
        _mlp_final_body,
        grid=(N_NODES_C // _BN,),
        in_specs=[_row_spec(), _row_spec(), _row_spec(),
                  _full_spec((D_C, D_C)), _full_spec((1, D_C)),
                  _full_spec((1, 1))],
        out_specs=(_row_spec(), _row_spec()),
        out_shape=(jax.ShapeDtypeStruct((N_NODES_C, D_C), jnp.float32),
                   jax.ShapeDtypeStruct((N_NODES_C, D_C), jnp.float32)),
    )(h, a0, a1, W, b.reshape(1, D_C), eps.reshape(1, 1))


def kernel(x, edge_index, W1, b1, eps1, W2, b2, eps2, W3, b3, eps3):
    pad = _EPW_PAD - _EPW
    src = jnp.pad(edge_index[0].astype(jnp.int32).reshape(_NW, _EPW),
                  ((0, 0), (0, pad))).reshape(-1)
    dst = jnp.pad(edge_index[1].astype(jnp.int32).reshape(_NW, _EPW),
                  ((0, 0), (0, pad)),
                  constant_values=N_NODES_C).reshape(-1)
    zeros = jnp.zeros((N_NODES_C, D_C), jnp.float32)

    agg = _sc_agg(src, dst, x, zeros)
    h = _tc_mlp(x, agg[0], agg[1], W1, b1, eps1)
    agg = _sc_agg(src, dst, h, zeros)
    h = _tc_mlp(h, agg[0], agg[1], W2, b2, eps2)
    agg = _sc_agg(src, dst, h, zeros)
    logits, probs = _tc_mlp_final(h, agg[0], agg[1], W3, b3, eps3)
    return (logits, probs)
